# baseline (speedup 1.0000x reference)
import jax
import jax.numpy as jnp
import numpy as np
from jax import lax
from jax.experimental import pallas as pl
from jax.experimental.pallas import tpu as pltpu
from jax.experimental.pallas import tpu_sc as plsc

EMB_DIM = 128
BATCH_N = 16384
N_CORES = 2
N_SUBCORES = 16
LANES = 16
SEGS = EMB_DIM // LANES
N_WORKERS = N_CORES * N_SUBCORES
ROWS_PER_WORKER = BATCH_N // N_WORKERS
CHUNK = 64
N_CHUNKS = ROWS_PER_WORKER // CHUNK
GROUPS = CHUNK // LANES
PACKED_W = EMB_DIM // 2


def _pack_bf16_table(w):
    w4 = lax.bitcast_convert_type(
        w.reshape(w.shape[0], EMB_DIM // 32, 2, 16), jnp.int32)
    lo = jax.lax.shift_right_logical(w4[:, :, 0, :], 16)
    hi = jnp.bitwise_and(w4[:, :, 1, :], jnp.int32(-65536))
    return jnp.bitwise_or(lo, hi).reshape(w.shape[0], PACKED_W)


def _sc_body(h_hbm, t_hbm, r_hbm, ent_hbm, reln_hbm, out_hbm,
             hi_all, ti_all, ri_all,
             hr2, tr2, rn2, score_all, sems):
    wid = lax.axis_index("s") * N_CORES + lax.axis_index("c")
    base = wid * ROWS_PER_WORKER
    lanes = lax.iota(jnp.int32, LANES)

    pltpu.sync_copy(h_hbm.at[pl.ds(base, ROWS_PER_WORKER)], hi_all)
    pltpu.sync_copy(t_hbm.at[pl.ds(base, ROWS_PER_WORKER)], ti_all)
    pltpu.sync_copy(r_hbm.at[pl.ds(base, ROWS_PER_WORKER)], ri_all)

    def fire(ck, par):
        sl = pl.ds(ck * CHUNK, CHUNK)
        sem = sems.at[par]
        pltpu.async_copy(ent_hbm.at[hi_all.at[sl]], hr2.at[par], sem)
        pltpu.async_copy(ent_hbm.at[ti_all.at[sl]], tr2.at[par], sem)
        pltpu.async_copy(reln_hbm.at[ri_all.at[sl]], rn2.at[par], sem)

    def wait3(par):
        sl = pl.ds(0, CHUNK)
        sem = sems.at[par]
        pltpu.make_async_copy(ent_hbm.at[hi_all.at[sl]], hr2.at[par],
                              sem).wait()
        pltpu.make_async_copy(ent_hbm.at[ti_all.at[sl]], tr2.at[par],
                              sem).wait()
        pltpu.make_async_copy(reln_hbm.at[ri_all.at[sl]], rn2.at[par],
                              sem).wait()

    fire(0, 0)

    def chunk_body(ck, carry):
        par = jnp.bitwise_and(ck, 1)

        @pl.when(ck < N_CHUNKS - 1)
        def _():
            fire(ck + 1, 1 - par)

        wait3(par)
        lane0 = lanes == 0

        def row_body(i, carry2):
            d = [hr2[par, i, pl.ds(j * LANES, LANES)]
                 - tr2[par, i, pl.ds(j * LANES, LANES)]
                 for j in range(SEGS)]
            n = []
            for blk in range(SEGS // 2):
                nw = rn2[par, i, pl.ds(PACKED_W + blk * LANES, LANES)]
                n.append(plsc.bitcast(
                    lax.shift_left(nw, 16), jnp.float32))
                n.append(plsc.bitcast(
                    jnp.bitwise_and(nw, jnp.int32(-65536)), jnp.float32))
            dot = d[0] * n[0]
            for j in range(1, SEGS):
                dot = dot + d[j] * n[j]
            u = []
            for blk in range(SEGS // 2):
                rw = rn2[par, i, pl.ds(blk * LANES, LANES)]
                u.append(d[2 * blk] + plsc.bitcast(
                    lax.shift_left(rw, 16), jnp.float32))
                u.append(d[2 * blk + 1] + plsc.bitcast(
                    jnp.bitwise_and(rw, jnp.int32(-65536)), jnp.float32))
            s = jnp.sum(dot)
            acc = jnp.abs(u[0] - s * n[0])
            for j in range(1, SEGS):
                acc = acc + jnp.abs(u[j] - s * n[j])
            sc = jnp.sum(acc)
            idx = jnp.full((LANES,), 0, jnp.int32) + (ck * CHUNK + i)
            plsc.store_scatter(score_all, [idx],
                               jnp.zeros((LANES,), jnp.float32) + sc,
                               mask=lane0)
            return carry2

        lax.fori_loop(0, CHUNK, row_body, 0)
        return carry

    lax.fori_loop(0, N_CHUNKS, chunk_body, 0)
    pltpu.sync_copy(score_all, out_hbm.at[pl.ds(base, ROWS_PER_WORKER)])


def kernel(h, t, r, ent_weight, rel_weight, norm_weight):
    mesh = plsc.VectorSubcoreMesh(core_axis_name="c", subcore_axis_name="s")
    run = pl.kernel(
        _sc_body,
        out_type=jax.ShapeDtypeStruct((BATCH_N,), jnp.float32),
        mesh=mesh,
        compiler_params=pltpu.CompilerParams(needs_layout_passes=False),
        scratch_types=[
            pltpu.VMEM((ROWS_PER_WORKER,), jnp.int32),
            pltpu.VMEM((ROWS_PER_WORKER,), jnp.int32),
            pltpu.VMEM((ROWS_PER_WORKER,), jnp.int32),
            pltpu.VMEM((2, CHUNK, EMB_DIM), jnp.float32),
            pltpu.VMEM((2, CHUNK, EMB_DIM), jnp.float32),
            pltpu.VMEM((2, CHUNK, 2 * PACKED_W), jnp.int32),
            pltpu.VMEM((ROWS_PER_WORKER,), jnp.float32),
            pltpu.SemaphoreType.DMA((2,)),
        ],
    )
    reln = jnp.concatenate(
        [_pack_bf16_table(rel_weight), _pack_bf16_table(norm_weight)], axis=1)
    return run(h.astype(jnp.int32), t.astype(jnp.int32), r.astype(jnp.int32),
               ent_weight, reln)


# speedup vs baseline: 1.2936x; 1.2936x over previous
import jax
import jax.numpy as jnp
import numpy as np
from jax import lax
from jax.experimental import pallas as pl
from jax.experimental.pallas import tpu as pltpu
from jax.experimental.pallas import tpu_sc as plsc

EMB_DIM = 128
BATCH_N = 16384
N_CORES = 2
N_SUBCORES = 16
LANES = 16
SEGS = EMB_DIM // LANES
N_WORKERS = N_CORES * N_SUBCORES
ROWS_PER_WORKER = BATCH_N // N_WORKERS
CHUNK = 64
N_CHUNKS = ROWS_PER_WORKER // CHUNK
GROUPS = CHUNK // LANES
PACKED_W = EMB_DIM // 2


def _pack_bf16_table(w):
    w4 = lax.bitcast_convert_type(
        w.reshape(w.shape[0], EMB_DIM // 32, 2, 16), jnp.int32)
    lo = jax.lax.shift_right_logical(w4[:, :, 0, :], 16)
    hi = jnp.bitwise_and(w4[:, :, 1, :], jnp.int32(-65536))
    return jnp.bitwise_or(lo, hi).reshape(w.shape[0], PACKED_W)


def _sc_body(h_hbm, t_hbm, r_hbm, ent_hbm, reln_hbm, out_hbm,
             hi_all, ti_all, ri_all,
             hr2, tr2, rn2, score_all, sems):
    wid = lax.axis_index("s") * N_CORES + lax.axis_index("c")
    base = wid * ROWS_PER_WORKER
    lanes = lax.iota(jnp.int32, LANES)

    pltpu.sync_copy(h_hbm.at[pl.ds(base, ROWS_PER_WORKER)], hi_all)
    pltpu.sync_copy(t_hbm.at[pl.ds(base, ROWS_PER_WORKER)], ti_all)
    pltpu.sync_copy(r_hbm.at[pl.ds(base, ROWS_PER_WORKER)], ri_all)

    def fire(ck, par):
        sl = pl.ds(ck * CHUNK, CHUNK)
        sem = sems.at[par]
        pltpu.async_copy(ent_hbm.at[hi_all.at[sl]], hr2.at[par], sem)
        pltpu.async_copy(ent_hbm.at[ti_all.at[sl]], tr2.at[par], sem)
        pltpu.async_copy(reln_hbm.at[ri_all.at[sl]], rn2.at[par], sem)

    def wait3(par):
        sl = pl.ds(0, CHUNK)
        sem = sems.at[par]
        pltpu.make_async_copy(ent_hbm.at[hi_all.at[sl]], hr2.at[par],
                              sem).wait()
        pltpu.make_async_copy(ent_hbm.at[ti_all.at[sl]], tr2.at[par],
                              sem).wait()
        pltpu.make_async_copy(reln_hbm.at[ri_all.at[sl]], rn2.at[par],
                              sem).wait()

    fire(0, 0)

    def chunk_body(ck, carry):
        par = jnp.bitwise_and(ck, 1)

        @pl.when(ck < N_CHUNKS - 1)
        def _():
            fire(ck + 1, 1 - par)

        wait3(par)
        lane0 = lanes == 0

        @plsc.parallel_loop(0, CHUNK, step=1, unroll=4)
        def row_body(i):
            d = [hr2[par, i, pl.ds(j * LANES, LANES)]
                 - tr2[par, i, pl.ds(j * LANES, LANES)]
                 for j in range(SEGS)]
            n = []
            for blk in range(SEGS // 2):
                nw = rn2[par, i, pl.ds(PACKED_W + blk * LANES, LANES)]
                n.append(plsc.bitcast(
                    lax.shift_left(nw, 16), jnp.float32))
                n.append(plsc.bitcast(
                    jnp.bitwise_and(nw, jnp.int32(-65536)), jnp.float32))
            dot = d[0] * n[0]
            for j in range(1, SEGS):
                dot = dot + d[j] * n[j]
            u = []
            for blk in range(SEGS // 2):
                rw = rn2[par, i, pl.ds(blk * LANES, LANES)]
                u.append(d[2 * blk] + plsc.bitcast(
                    lax.shift_left(rw, 16), jnp.float32))
                u.append(d[2 * blk + 1] + plsc.bitcast(
                    jnp.bitwise_and(rw, jnp.int32(-65536)), jnp.float32))
            s = jnp.sum(dot)
            acc = jnp.abs(u[0] - s * n[0])
            for j in range(1, SEGS):
                acc = acc + jnp.abs(u[j] - s * n[j])
            sc = jnp.sum(acc)
            idx = jnp.full((LANES,), 0, jnp.int32) + (ck * CHUNK + i)
            plsc.store_scatter(score_all, [idx],
                               jnp.zeros((LANES,), jnp.float32) + sc,
                               mask=lane0)

        return carry

    lax.fori_loop(0, N_CHUNKS, chunk_body, 0)
    pltpu.sync_copy(score_all, out_hbm.at[pl.ds(base, ROWS_PER_WORKER)])


def kernel(h, t, r, ent_weight, rel_weight, norm_weight):
    mesh = plsc.VectorSubcoreMesh(core_axis_name="c", subcore_axis_name="s")
    run = pl.kernel(
        _sc_body,
        out_type=jax.ShapeDtypeStruct((BATCH_N,), jnp.float32),
        mesh=mesh,
        compiler_params=pltpu.CompilerParams(needs_layout_passes=False),
        scratch_types=[
            pltpu.VMEM((ROWS_PER_WORKER,), jnp.int32),
            pltpu.VMEM((ROWS_PER_WORKER,), jnp.int32),
            pltpu.VMEM((ROWS_PER_WORKER,), jnp.int32),
            pltpu.VMEM((2, CHUNK, EMB_DIM), jnp.float32),
            pltpu.VMEM((2, CHUNK, EMB_DIM), jnp.float32),
            pltpu.VMEM((2, CHUNK, 2 * PACKED_W), jnp.int32),
            pltpu.VMEM((ROWS_PER_WORKER,), jnp.float32),
            pltpu.SemaphoreType.DMA((2,)),
        ],
    )
    reln = jnp.concatenate(
        [_pack_bf16_table(rel_weight), _pack_bf16_table(norm_weight)], axis=1)
    return run(h.astype(jnp.int32), t.astype(jnp.int32), r.astype(jnp.int32),
               ent_weight, reln)
